# baseline (speedup 1.0000x reference)
import functools

import jax
import jax.numpy as jnp
from jax import lax
from jax.experimental import pallas as pl
from jax.experimental.pallas import tpu as pltpu
from jax.experimental.pallas import tpu_sc as plsc

N = 16384
K = 8192
D = 256
BN = 2048
BK = 512
LOSS_SCALE = 1.25 / (N * D)


def _row_norm(x):
    p = x * x
    c = p[:, :128] + p[:, 128:]
    acc = c[:, 0:8]
    for g in range(1, 16):
        acc = acc + c[:, 8 * g:8 * g + 8]
    a4 = acc[:, 0:4] + acc[:, 4:8]
    a2 = a4[:, 0:2] + a4[:, 2:4]
    return a2[:, 0:1] + a2[:, 1:2]


def _argmin_body(z_ref, e_ref, idx_ref, loss_ref, m_ref, i_ref):
    i = pl.program_id(0)
    z = z_ref[...]
    z2 = _row_norm(z)
    zs = z + z

    def chunk_dist(kk):
        e = e_ref[pl.ds(kk * BK, BK), :]
        e2 = _row_norm(e)[:, 0][None, :]
        dot2 = lax.dot_general(zs, e, (((1,), (1,)), ((), ())),
                               preferred_element_type=jnp.float32)
        return jnp.sqrt(jnp.maximum((z2 + e2) - dot2, 0.0))

    m_ref[...] = chunk_dist(0)
    i_ref[...] = jnp.zeros((BN, BK), jnp.int32)

    def step(kk, _):
        dist = chunk_dist(kk)
        better = dist < m_ref[...]
        m_ref[...] = jnp.where(better, dist, m_ref[...])
        i_ref[...] = jnp.where(better, kk, i_ref[...])
        return 0

    lax.fori_loop(1, K // BK, step, 0)
    M = m_ref[...]
    mn = jnp.min(M, axis=1, keepdims=True)
    gid = i_ref[...] * BK + lax.broadcasted_iota(jnp.int32, (BN, BK), 1)
    idx_ref[...] = jnp.min(jnp.where(M == mn, gid, K), axis=1)
    part = (jnp.sum(mn * mn) * LOSS_SCALE).reshape(1, 1)

    @pl.when(i == 0)
    def _():
        loss_ref[...] = part

    @pl.when(i > 0)
    def _():
        loss_ref[...] = loss_ref[...] + part


_argmin_call = pl.pallas_call(
    _argmin_body,
    grid=(N // BN,),
    in_specs=[
        pl.BlockSpec((BN, D), lambda i: (i, 0)),
        pl.BlockSpec((K, D), lambda i: (0, 0)),
    ],
    out_specs=[
        pl.BlockSpec((BN,), lambda i: (i,)),
        pl.BlockSpec((1, 1), lambda i: (0, 0)),
    ],
    out_shape=[
        jax.ShapeDtypeStruct((N,), jnp.int32),
        jax.ShapeDtypeStruct((1, 1), jnp.float32),
    ],
    scratch_shapes=[
        pltpu.VMEM((BN, BK), jnp.float32),
        pltpu.VMEM((BN, BK), jnp.int32),
    ],
)

_NW = 32
_BPW = N // _NW
_CH = 128


@functools.cache
def _sc_gather():
    @functools.partial(
        pl.kernel,
        mesh=plsc.VectorSubcoreMesh(core_axis_name="c", subcore_axis_name="s"),
        out_type=jax.ShapeDtypeStruct((N, D), jnp.float32),
        scratch_types=[
            pltpu.VMEM((_CH,), jnp.int32),
            pltpu.VMEM((_CH, D), jnp.float32),
            pltpu.SemaphoreType.DMA,
        ],
    )
    def gather(table_hbm, idx_hbm, out_hbm, idx_v, rows_v, sem):
        wid = lax.axis_index("s") * 2 + lax.axis_index("c")
        base = wid * _BPW
        for c in range(_BPW // _CH):
            o = base + c * _CH
            pltpu.sync_copy(idx_hbm.at[pl.ds(o, _CH)], idx_v)
            pltpu.async_copy(table_hbm.at[idx_v], rows_v, sem).wait()
            pltpu.sync_copy(rows_v, out_hbm.at[pl.ds(o, _CH)])

    return gather


def kernel(z, embeddings):
    idx, loss = _argmin_call(z, embeddings)
    quantized = _sc_gather()(embeddings, idx)
    return idx, quantized, loss[0, 0]


# speedup vs baseline: 2.3224x; 2.3224x over previous
import functools

import jax
import jax.numpy as jnp
from jax import lax
from jax.experimental import pallas as pl
from jax.experimental.pallas import tpu as pltpu
from jax.experimental.pallas import tpu_sc as plsc

N = 16384
K = 8192
D = 256
BN = 2048
BK = 512
LOSS_SCALE = 1.25 / (N * D)


def _row_norm_t(x):
    p = x * x
    c = p[:, :128] + p[:, 128:]
    t = c.T
    acc = t[0:8, :]
    for g in range(1, 16):
        acc = acc + t[8 * g:8 * g + 8, :]
    a4 = acc[0:4, :] + acc[4:8, :]
    a2 = a4[0:2, :] + a4[2:4, :]
    return a2[0:1, :] + a2[1:2, :]


def _argmin_body(z_ref, e_ref, idx_ref, loss_ref, m_ref, i_ref, e2_ref):
    i = pl.program_id(0)
    z = z_ref[...]
    z2 = _row_norm_t(z).T
    zs = z + z

    @pl.when(i == 0)
    def _():
        def einit(kk, _):
            e = e_ref[pl.ds(kk * BK, BK), :]
            e2_ref[0:1, pl.ds(kk * BK, BK)] = _row_norm_t(e)
            return 0
        lax.fori_loop(0, K // BK, einit, 0, unroll=True)

    def chunk_dist(kk):
        e = e_ref[pl.ds(kk * BK, BK), :]
        e2 = e2_ref[0:1, pl.ds(kk * BK, BK)]
        dot2 = lax.dot_general(zs, e, (((1,), (1,)), ((), ())),
                               preferred_element_type=jnp.float32)
        return jnp.sqrt(jnp.maximum((z2 + e2) - dot2, 0.0))

    m_ref[...] = chunk_dist(0)
    i_ref[...] = jnp.zeros((BN, BK), jnp.int32)

    def step(kk, _):
        dist = chunk_dist(kk)
        better = dist < m_ref[...]
        m_ref[...] = jnp.where(better, dist, m_ref[...])
        i_ref[...] = jnp.where(better, kk, i_ref[...])
        return 0

    lax.fori_loop(1, K // BK, step, 0)
    M = m_ref[...]
    mn = jnp.min(M, axis=1, keepdims=True)
    gid = i_ref[...] * BK + lax.broadcasted_iota(jnp.int32, (BN, BK), 1)
    idx_ref[...] = jnp.min(jnp.where(M == mn, gid, K), axis=1)
    part = (jnp.sum(mn * mn) * LOSS_SCALE).reshape(1, 1)

    @pl.when(i == 0)
    def _():
        loss_ref[...] = part

    @pl.when(i > 0)
    def _():
        loss_ref[...] = loss_ref[...] + part


_argmin_call = pl.pallas_call(
    _argmin_body,
    grid=(N // BN,),
    in_specs=[
        pl.BlockSpec((BN, D), lambda i: (i, 0)),
        pl.BlockSpec((K, D), lambda i: (0, 0)),
    ],
    out_specs=[
        pl.BlockSpec((BN,), lambda i: (i,)),
        pl.BlockSpec((1, 1), lambda i: (0, 0)),
    ],
    out_shape=[
        jax.ShapeDtypeStruct((N,), jnp.int32),
        jax.ShapeDtypeStruct((1, 1), jnp.float32),
    ],
    scratch_shapes=[
        pltpu.VMEM((BN, BK), jnp.float32),
        pltpu.VMEM((BN, BK), jnp.int32),
        pltpu.VMEM((1, K), jnp.float32),
    ],
)

_NW = 32
_BPW = N // _NW
_CH = 128


@functools.cache
def _sc_gather():
    @functools.partial(
        pl.kernel,
        mesh=plsc.VectorSubcoreMesh(core_axis_name="c", subcore_axis_name="s"),
        out_type=jax.ShapeDtypeStruct((N, D), jnp.float32),
        scratch_types=[
            pltpu.VMEM((_CH,), jnp.int32),
            pltpu.VMEM((_CH, D), jnp.float32),
            pltpu.SemaphoreType.DMA,
        ],
    )
    def gather(table_hbm, idx_hbm, out_hbm, idx_v, rows_v, sem):
        wid = lax.axis_index("s") * 2 + lax.axis_index("c")
        base = wid * _BPW
        for c in range(_BPW // _CH):
            o = base + c * _CH
            pltpu.sync_copy(idx_hbm.at[pl.ds(o, _CH)], idx_v)
            pltpu.async_copy(table_hbm.at[idx_v], rows_v, sem).wait()
            pltpu.sync_copy(rows_v, out_hbm.at[pl.ds(o, _CH)])

    return gather


def kernel(z, embeddings):
    idx, loss = _argmin_call(z, embeddings)
    quantized = _sc_gather()(embeddings, idx)
    return idx, quantized, loss[0, 0]


# speedup vs baseline: 3.4515x; 1.4862x over previous
import functools

import jax
import jax.numpy as jnp
from jax import lax
from jax.experimental import pallas as pl
from jax.experimental.pallas import tpu as pltpu
from jax.experimental.pallas import tpu_sc as plsc

N = 16384
K = 8192
D = 256
BN = 2048
BK = 512
LOSS_SCALE = 1.25 / (N * D)


def _row_norm_t(x):
    p = x * x
    c = p[:, :128] + p[:, 128:]
    t = c.T
    acc = t[0:8, :]
    for g in range(1, 16):
        acc = acc + t[8 * g:8 * g + 8, :]
    a4 = acc[0:4, :] + acc[4:8, :]
    a2 = a4[0:2, :] + a4[2:4, :]
    return a2[0:1, :] + a2[1:2, :]


def _argmin_body(z_ref, e_ref, idx_ref, loss_ref, m_ref, i_ref, e2_ref):
    i = pl.program_id(0)
    z = z_ref[...]
    z2 = _row_norm_t(z).T
    zs = z + z

    @pl.when(i == 0)
    def _():
        def einit(kk, _):
            e = e_ref[pl.ds(kk * BK, BK), :]
            e2_ref[0:1, pl.ds(kk * BK, BK)] = _row_norm_t(e)
            return 0
        lax.fori_loop(0, K // BK, einit, 0, unroll=True)

    def chunk_d2(kk):
        e = e_ref[pl.ds(kk * BK, BK), :]
        e2 = e2_ref[0:1, pl.ds(kk * BK, BK)]
        dot2 = lax.dot_general(zs, e, (((1,), (1,)), ((), ())),
                               preferred_element_type=jnp.float32)
        return jnp.maximum((z2 + e2) - dot2, 0.0)

    m_ref[...] = chunk_d2(0)
    i_ref[...] = jnp.zeros((BN, BK), jnp.float32)

    def step(kk, _):
        d2 = chunk_d2(kk)
        better = d2 < m_ref[...]
        m_ref[...] = jnp.where(better, d2, m_ref[...])
        i_ref[...] = jnp.where(better, kk.astype(jnp.float32), i_ref[...])
        return 0

    lax.fori_loop(1, K // BK, step, 0)
    dist = jnp.sqrt(m_ref[...])
    mn = jnp.min(dist, axis=1, keepdims=True)
    lane = lax.broadcasted_iota(jnp.int32, (BN, BK), 1).astype(jnp.float32)
    gid = i_ref[...] * float(BK) + lane
    am = jnp.min(jnp.where(dist == mn, gid, float(K)), axis=1)
    idx_ref[...] = am.astype(jnp.int32)
    part = (jnp.sum(mn * mn) * LOSS_SCALE).reshape(1, 1)

    @pl.when(i == 0)
    def _():
        loss_ref[...] = part

    @pl.when(i > 0)
    def _():
        loss_ref[...] = loss_ref[...] + part


_argmin_call = pl.pallas_call(
    _argmin_body,
    grid=(N // BN,),
    in_specs=[
        pl.BlockSpec((BN, D), lambda i: (i, 0)),
        pl.BlockSpec((K, D), lambda i: (0, 0)),
    ],
    out_specs=[
        pl.BlockSpec((BN,), lambda i: (i,)),
        pl.BlockSpec((1, 1), lambda i: (0, 0)),
    ],
    out_shape=[
        jax.ShapeDtypeStruct((N,), jnp.int32),
        jax.ShapeDtypeStruct((1, 1), jnp.float32),
    ],
    scratch_shapes=[
        pltpu.VMEM((BN, BK), jnp.float32),
        pltpu.VMEM((BN, BK), jnp.float32),
        pltpu.VMEM((1, K), jnp.float32),
    ],
)

_NW = 32
_BPW = N // _NW
_CH = 128


@functools.cache
def _sc_gather():
    @functools.partial(
        pl.kernel,
        mesh=plsc.VectorSubcoreMesh(core_axis_name="c", subcore_axis_name="s"),
        out_type=jax.ShapeDtypeStruct((N, D), jnp.float32),
        scratch_types=[
            pltpu.VMEM((_CH,), jnp.int32),
            pltpu.VMEM((_CH, D), jnp.float32),
            pltpu.SemaphoreType.DMA,
        ],
    )
    def gather(table_hbm, idx_hbm, out_hbm, idx_v, rows_v, sem):
        wid = lax.axis_index("s") * 2 + lax.axis_index("c")
        base = wid * _BPW
        for c in range(_BPW // _CH):
            o = base + c * _CH
            pltpu.sync_copy(idx_hbm.at[pl.ds(o, _CH)], idx_v)
            pltpu.async_copy(table_hbm.at[idx_v], rows_v, sem).wait()
            pltpu.sync_copy(rows_v, out_hbm.at[pl.ds(o, _CH)])

    return gather


def kernel(z, embeddings):
    idx, loss = _argmin_call(z, embeddings)
    quantized = _sc_gather()(embeddings, idx)
    return idx, quantized, loss[0, 0]


# speedup vs baseline: 3.6348x; 1.0531x over previous
import functools

import jax
import jax.numpy as jnp
from jax import lax
from jax.experimental import pallas as pl
from jax.experimental.pallas import tpu as pltpu
from jax.experimental.pallas import tpu_sc as plsc

N = 16384
K = 8192
D = 256
BN = 2048
BK = 512
LOSS_SCALE = 1.25 / (N * D)


def _row_norm_t(x):
    p = x * x
    c = p[:, :128] + p[:, 128:]
    t = c.T
    acc = t[0:8, :]
    for g in range(1, 16):
        acc = acc + t[8 * g:8 * g + 8, :]
    a4 = acc[0:4, :] + acc[4:8, :]
    a2 = a4[0:2, :] + a4[2:4, :]
    return a2[0:1, :] + a2[1:2, :]


def _argmin_body(z_ref, e_ref, idx_ref, loss_ref, m_ref, i_ref, e2_ref):
    i = pl.program_id(0)
    z = z_ref[...]
    z2 = _row_norm_t(z).T
    zs = z + z

    @pl.when(i == 0)
    def _():
        def einit(kk, _):
            e = e_ref[pl.ds(kk * BK, BK), :]
            e2_ref[0:1, pl.ds(kk * BK, BK)] = _row_norm_t(e)
            return 0
        lax.fori_loop(0, K // BK, einit, 0, unroll=True)

    def chunk_d2(kk):
        e = e_ref[pl.ds(kk * BK, BK), :]
        e2 = e2_ref[0:1, pl.ds(kk * BK, BK)]
        dot2 = lax.dot_general(zs, e, (((1,), (1,)), ((), ())),
                               preferred_element_type=jnp.float32)
        return (z2 + e2) - dot2

    m_ref[...] = chunk_d2(0)
    i_ref[...] = jnp.zeros((BN, BK), jnp.float32)

    def step(kk, _):
        d2 = chunk_d2(kk)
        better = d2 < m_ref[...]
        m_ref[...] = jnp.where(better, d2, m_ref[...])
        i_ref[...] = jnp.where(better, kk.astype(jnp.float32), i_ref[...])
        return 0

    lax.fori_loop(1, K // BK, step, 0, unroll=4)
    dist = jnp.sqrt(jnp.maximum(m_ref[...], 0.0))
    mn = jnp.min(dist, axis=1, keepdims=True)
    lane = lax.broadcasted_iota(jnp.int32, (BN, BK), 1).astype(jnp.float32)
    gid = i_ref[...] * float(BK) + lane
    am = jnp.min(jnp.where(dist == mn, gid, float(K)), axis=1)
    idx_ref[...] = am.astype(jnp.int32)
    part = (jnp.sum(mn * mn) * LOSS_SCALE).reshape(1, 1)

    @pl.when(i == 0)
    def _():
        loss_ref[...] = part

    @pl.when(i > 0)
    def _():
        loss_ref[...] = loss_ref[...] + part


_argmin_call = pl.pallas_call(
    _argmin_body,
    grid=(N // BN,),
    in_specs=[
        pl.BlockSpec((BN, D), lambda i: (i, 0)),
        pl.BlockSpec((K, D), lambda i: (0, 0)),
    ],
    out_specs=[
        pl.BlockSpec((BN,), lambda i: (i,)),
        pl.BlockSpec((1, 1), lambda i: (0, 0)),
    ],
    out_shape=[
        jax.ShapeDtypeStruct((N,), jnp.int32),
        jax.ShapeDtypeStruct((1, 1), jnp.float32),
    ],
    scratch_shapes=[
        pltpu.VMEM((BN, BK), jnp.float32),
        pltpu.VMEM((BN, BK), jnp.float32),
        pltpu.VMEM((1, K), jnp.float32),
    ],
)

_NW = 32
_BPW = N // _NW
_CH = 128


@functools.cache
def _sc_gather():
    @functools.partial(
        pl.kernel,
        mesh=plsc.VectorSubcoreMesh(core_axis_name="c", subcore_axis_name="s"),
        out_type=jax.ShapeDtypeStruct((N, D), jnp.float32),
        scratch_types=[
            pltpu.VMEM((_CH,), jnp.int32),
            pltpu.VMEM((_CH, D), jnp.float32),
            pltpu.SemaphoreType.DMA,
        ],
    )
    def gather(table_hbm, idx_hbm, out_hbm, idx_v, rows_v, sem):
        wid = lax.axis_index("s") * 2 + lax.axis_index("c")
        base = wid * _BPW
        for c in range(_BPW // _CH):
            o = base + c * _CH
            pltpu.sync_copy(idx_hbm.at[pl.ds(o, _CH)], idx_v)
            pltpu.async_copy(table_hbm.at[idx_v], rows_v, sem).wait()
            pltpu.sync_copy(rows_v, out_hbm.at[pl.ds(o, _CH)])

    return gather


def kernel(z, embeddings):
    idx, loss = _argmin_call(z, embeddings)
    quantized = _sc_gather()(embeddings, idx)
    return idx, quantized, loss[0, 0]
